# baseline (speedup 1.0000x reference)
import functools

import jax
import jax.numpy as jnp
from jax import lax
from jax.experimental import pallas as pl
from jax.experimental.pallas import tpu as pltpu
from jax.experimental.pallas import tpu_sc as plsc


_KC = 128


def _dist_argmin_body(xt_ref, cb_ref, z_ref, acc_ref, b2_ref, rowf_ref):
    i = pl.program_id(0)

    @pl.when(i == 0)
    def _init():
        cb0 = cb_ref[...]
        b2_ref[...] = jnp.sum(cb0 * cb0, axis=1, keepdims=True)
        rowf_ref[...] = lax.broadcasted_iota(
            jnp.int32, rowf_ref.shape, 0).astype(jnp.float32)
        acc_ref[0, 0] = 0.0

    xtb = xt_ref[...]
    xs = xtb * -2.0
    a2 = jnp.sum(xtb * xtb, axis=0, keepdims=True)
    k = cb_ref.shape[0]
    runval = None
    runcf = None
    for c in range(k // _KC):
        cbc = cb_ref[c * _KC:(c + 1) * _KC, :]
        m = lax.dot_general(cbc, xs, (((1,), (0,)), ((), ())),
                            preferred_element_type=jnp.float32)
        d = (a2 + b2_ref[c * _KC:(c + 1) * _KC, :]) + m
        if c == 0:
            runval = d
            runcf = jnp.zeros_like(d)
        else:
            better = d < runval
            runval = jnp.where(better, d, runval)
            runcf = jnp.where(better, jnp.float32(c), runcf)
    minv = jnp.min(runval, axis=0, keepdims=True)
    kcand = runcf * jnp.float32(_KC) + rowf_ref[...]
    zf = jnp.min(jnp.where(runval == minv, kcand, jnp.float32(k)), axis=0)
    z_ref[...] = zf.astype(jnp.int32)
    acc_ref[0, 0] += jnp.sum(minv)


def _dist_argmin(xt, codebook, block_cols):
    d, n = xt.shape
    k = codebook.shape[0]
    return pl.pallas_call(
        _dist_argmin_body,
        grid=(n // block_cols,),
        in_specs=[
            pl.BlockSpec((d, block_cols), lambda i: (0, i)),
            pl.BlockSpec((k, d), lambda i: (0, 0)),
        ],
        out_specs=[
            pl.BlockSpec((block_cols,), lambda i: (i,)),
            pl.BlockSpec(memory_space=pltpu.SMEM),
        ],
        out_shape=[
            jax.ShapeDtypeStruct((n,), jnp.int32),
            jax.ShapeDtypeStruct((1, 1), jnp.float32),
        ],
        scratch_shapes=[pltpu.VMEM((k, 1), jnp.float32),
                        pltpu.VMEM((_KC, 1), jnp.float32)],
    )(xt, codebook)


@functools.cache
def _make_sc_gather(v, d, b, dtype):
    info = plsc.get_sparse_core_info()
    nc, ns = info.num_cores, info.num_subcores
    nw = nc * ns
    b_per_w = b // nw
    mesh = plsc.VectorSubcoreMesh(core_axis_name="c", subcore_axis_name="s")

    @functools.partial(
        pl.kernel, mesh=mesh,
        compiler_params=pltpu.CompilerParams(use_tc_tiling_on_sc=False),
        out_type=jax.ShapeDtypeStruct((b, d), dtype),
        scratch_types=[
            pltpu.VMEM((b_per_w,), jnp.int32),
            pltpu.VMEM((b_per_w, d), dtype),
            pltpu.SemaphoreType.DMA,
        ],
    )
    def gather(table_hbm, idx_hbm, out_hbm, idx_v, rows_v, sem):
        wid = lax.axis_index("s") * nc + lax.axis_index("c")
        base = wid * b_per_w
        pltpu.sync_copy(idx_hbm.at[pl.ds(base, b_per_w)], idx_v)
        pltpu.async_copy(table_hbm.at[idx_v], rows_v, sem).wait()
        pltpu.sync_copy(rows_v, out_hbm.at[pl.ds(base, b_per_w)])

    return gather


def kernel(x, codebook):
    n, d = x.shape
    k = codebook.shape[0]
    z, acc = _dist_argmin(x.T, codebook, 2048)
    q = _make_sc_gather(k, d, n, codebook.dtype)(codebook, z)
    loss = acc[0, 0] / jnp.float32(n * d)
    return (z, q, loss, loss)


# speedup vs baseline: 1.0287x; 1.0287x over previous
import functools

import jax
import jax.numpy as jnp
from jax import lax
from jax.experimental import pallas as pl
from jax.experimental.pallas import tpu as pltpu
from jax.experimental.pallas import tpu_sc as plsc


_KC = 256


def _dist_argmin_body(xt_ref, cb_ref, z_ref, acc_ref, b2_ref, rowf_ref, *,
                      inv_count):
    i = pl.program_id(0)

    @pl.when(i == 0)
    def _init():
        cb0 = cb_ref[...]
        b2_ref[...] = jnp.sum(cb0 * cb0, axis=1, keepdims=True)
        rowf_ref[...] = lax.broadcasted_iota(
            jnp.int32, rowf_ref.shape, 0).astype(jnp.float32)
        acc_ref[0, 0] = 0.0

    xtb = xt_ref[...]
    xs = xtb * -2.0
    a2 = jnp.sum(xtb * xtb, axis=0, keepdims=True)
    k = cb_ref.shape[0]
    runval = None
    runcf = None
    for c in range(k // _KC):
        cbc = cb_ref[c * _KC:(c + 1) * _KC, :]
        m = lax.dot_general(cbc, xs, (((1,), (0,)), ((), ())),
                            preferred_element_type=jnp.float32)
        d = (a2 + b2_ref[c * _KC:(c + 1) * _KC, :]) + m
        if c == 0:
            runval = d
            runcf = jnp.zeros_like(d)
        else:
            better = d < runval
            runval = jnp.where(better, d, runval)
            runcf = jnp.where(better, jnp.float32(c), runcf)
    minv = jnp.min(runval, axis=0, keepdims=True)
    kcand = runcf * jnp.float32(_KC) + rowf_ref[...]
    zf = jnp.min(jnp.where(runval == minv, kcand, jnp.float32(k)), axis=0)
    z_ref[...] = zf.astype(jnp.int32)
    acc_ref[0, 0] += jnp.sum(minv)

    @pl.when(i == pl.num_programs(0) - 1)
    def _finish():
        acc_ref[0, 0] = acc_ref[0, 0] * inv_count


def _dist_argmin(xt, codebook, block_cols):
    d, n = xt.shape
    k = codebook.shape[0]
    return pl.pallas_call(
        functools.partial(_dist_argmin_body, inv_count=1.0 / (n * d)),
        grid=(n // block_cols,),
        in_specs=[
            pl.BlockSpec((d, block_cols), lambda i: (0, i)),
            pl.BlockSpec((k, d), lambda i: (0, 0)),
        ],
        out_specs=[
            pl.BlockSpec((block_cols,), lambda i: (i,)),
            pl.BlockSpec(memory_space=pltpu.SMEM),
        ],
        out_shape=[
            jax.ShapeDtypeStruct((n,), jnp.int32),
            jax.ShapeDtypeStruct((1, 1), jnp.float32),
        ],
        scratch_shapes=[pltpu.VMEM((k, 1), jnp.float32),
                        pltpu.VMEM((_KC, 1), jnp.float32)],
    )(xt, codebook)


@functools.cache
def _make_sc_gather(v, d, b, dtype):
    info = plsc.get_sparse_core_info()
    nc, ns = info.num_cores, info.num_subcores
    nw = nc * ns
    b_per_w = b // nw
    mesh = plsc.VectorSubcoreMesh(core_axis_name="c", subcore_axis_name="s")

    nchunk = 2
    cs = b_per_w // nchunk

    @functools.partial(
        pl.kernel, mesh=mesh,
        compiler_params=pltpu.CompilerParams(use_tc_tiling_on_sc=False),
        out_type=jax.ShapeDtypeStruct((b, d), dtype),
        scratch_types=(
            [pltpu.VMEM((cs,), jnp.int32)] * nchunk
            + [pltpu.VMEM((cs, d), dtype)] * nchunk
            + [pltpu.SemaphoreType.DMA] * (2 * nchunk)
        ),
    )
    def gather(table_hbm, idx_hbm, out_hbm, *scratch):
        idxs = scratch[:nchunk]
        rows = scratch[nchunk:2 * nchunk]
        sgs = scratch[2 * nchunk:3 * nchunk]
        sws = scratch[3 * nchunk:]
        wid = lax.axis_index("s") * nc + lax.axis_index("c")
        base = wid * b_per_w
        gs = []
        for j in range(nchunk):
            pltpu.sync_copy(idx_hbm.at[pl.ds(base + j * cs, cs)], idxs[j])
            gs.append(pltpu.async_copy(table_hbm.at[idxs[j]], rows[j], sgs[j]))
        ws = []
        for j in range(nchunk):
            gs[j].wait()
            ws.append(pltpu.async_copy(
                rows[j], out_hbm.at[pl.ds(base + j * cs, cs)], sws[j]))
        for w in ws:
            w.wait()

    return gather


def kernel(x, codebook):
    n, d = x.shape
    k = codebook.shape[0]
    z, acc = _dist_argmin(x.T, codebook, 2048)
    q = _make_sc_gather(k, d, n, codebook.dtype)(codebook, z)
    loss = acc[0, 0]
    return (z, q, loss, loss)
